# Initial kernel scaffold; baseline (speedup 1.0000x reference)
import jax
import jax.numpy as jnp
from jax.experimental import pallas as pl


def kernel(hidden_states, router_w, corr_bias, gate_w, up_w, down_w, s_gate, s_up, s_down):
    raise NotImplementedError("write your pallas kernel here")


# speedup vs baseline: 1.7394x; 1.7394x over previous
import functools

import jax
import jax.numpy as jnp
from jax.experimental import pallas as pl
from jax.experimental.pallas import tpu as pltpu

N, H = 2048, 1024
E, K, G, TG = 64, 8, 8, 4
EPG = E // G
F, FS = 256, 512
ROUTED_SCALE = 2.5
NEG = -1e30


def _first_max_mask(a):
    m = jnp.max(a, axis=1, keepdims=True)
    ids = jax.lax.broadcasted_iota(jnp.int32, a.shape, 1)
    cand = jnp.where(a == m, ids, a.shape[1])
    fid = jnp.min(cand, axis=1, keepdims=True)
    return ids == fid


def _moe_body(x_ref, rw_ref, cb_ref, gw_ref, uw_ref, dw_ref,
              sg_ref, su_ref, sd_ref, out_ref, comb_ref):
    e = pl.program_id(0)

    @pl.when(e == 0)
    def _gate_and_shared():
        x = x_ref[...]
        logits = jax.lax.dot_general(
            x, rw_ref[...], (((1,), (1,)), ((), ())),
            preferred_element_type=jnp.float32)
        scores = jax.nn.sigmoid(logits)
        sfc = scores + cb_ref[...]

        gs_cols = []
        for g in range(G):
            sub = sfc[:, g * EPG:(g + 1) * EPG]
            first = _first_max_mask(sub)
            m1 = jnp.max(sub, axis=1, keepdims=True)
            m2 = jnp.max(jnp.where(first, NEG, sub), axis=1, keepdims=True)
            gs_cols.append(m1 + m2)
        gs = jnp.concatenate(gs_cols, axis=1)

        gmask = jnp.zeros((N, G), dtype=jnp.float32)
        for _ in range(TG):
            first = _first_max_mask(jnp.where(gmask > 0, NEG, gs))
            gmask = gmask + first.astype(jnp.float32)
        emask_cols = [jnp.broadcast_to(gmask[:, g:g + 1], (N, EPG))
                      for g in range(G)]
        emask = jnp.concatenate(emask_cols, axis=1)

        tmp = jnp.where(emask > 0, sfc, NEG)
        sel = jnp.zeros((N, E), dtype=jnp.float32)
        for _ in range(K):
            first = _first_max_mask(jnp.where(sel > 0, NEG, tmp))
            sel = sel + first.astype(jnp.float32)

        w = sel * scores
        denom = jnp.sum(w, axis=1, keepdims=True) + 1e-20
        comb_ref[...] = w / denom * ROUTED_SCALE

        sg_ = jax.lax.dot_general(x, sg_ref[...], (((1,), (1,)), ((), ())),
                                  preferred_element_type=jnp.float32)
        su_ = jax.lax.dot_general(x, su_ref[...], (((1,), (1,)), ((), ())),
                                  preferred_element_type=jnp.float32)
        hs = su_ * (sg_ * jax.nn.sigmoid(sg_))
        sh = jax.lax.dot_general(hs, sd_ref[...], (((1,), (1,)), ((), ())),
                                 preferred_element_type=jnp.float32)
        out_ref[...] = x + sh

    x = x_ref[...]
    onehot = (jax.lax.broadcasted_iota(jnp.int32, (E, 1), 0) == e
              ).astype(jnp.float32)
    col = jax.lax.dot_general(comb_ref[...], onehot, (((1,), (0,)), ((), ())),
                              preferred_element_type=jnp.float32)
    g = jax.lax.dot_general(x, gw_ref[0], (((1,), (1,)), ((), ())),
                            preferred_element_type=jnp.float32)
    u = jax.lax.dot_general(x, uw_ref[0], (((1,), (1,)), ((), ())),
                            preferred_element_type=jnp.float32)
    h = (g * jax.nn.sigmoid(g)) * u * col
    y = jax.lax.dot_general(h, dw_ref[0], (((1,), (1,)), ((), ())),
                            preferred_element_type=jnp.float32)
    out_ref[...] += y


@functools.partial(jax.jit, static_argnames=("interpret",))
def _moe(x, rw, cb, gw, uw, dw, sg, su, sd, interpret=False):
    full = lambda shape: pl.BlockSpec(shape, lambda e: (0,) * len(shape))
    per_e = lambda shape: pl.BlockSpec(shape, lambda e: (e, 0, 0))
    return pl.pallas_call(
        _moe_body,
        grid=(E,),
        in_specs=[
            full((N, H)),
            full((E, H)),
            full((1, E)),
            per_e((1, F, H)),
            per_e((1, F, H)),
            per_e((1, H, F)),
            full((FS, H)),
            full((FS, H)),
            full((H, FS)),
        ],
        out_specs=full((N, H)),
        out_shape=jax.ShapeDtypeStruct((N, H), jnp.float32),
        scratch_shapes=[pltpu.VMEM((N, E), jnp.float32)],
        compiler_params=pltpu.CompilerParams(
            dimension_semantics=("arbitrary",),
        ),
        interpret=interpret,
    )(x, rw, cb, gw, uw, dw, sg, su, sd)


def kernel(hidden_states, router_w, corr_bias, gate_w, up_w, down_w,
           s_gate, s_up, s_down):
    Bq, Sq, Hq = hidden_states.shape
    x = hidden_states.reshape(N, H)
    y = _moe(x, router_w, corr_bias.reshape(1, E), gate_w, up_w, down_w,
             s_gate, s_up, s_down)
    return y.reshape(Bq, Sq, Hq)
